# baseline (speedup 1.0000x reference)
import functools

import jax
import jax.numpy as jnp
from jax import lax
from jax.experimental import pallas as pl
from jax.experimental.pallas import tpu as pltpu
from jax.experimental.pallas import tpu_sc as plsc

N = 50000
E = 800000
NC = 2
NS = 16
NW = NC * NS
IDX_B = 128
N_PAD = 50048
E_PAD = 819200
EROWS = E_PAD // IDX_B
RT4 = EROWS // NW
RT64 = EROWS // NS
DUMMY = N
CH = 8
ZR = N_PAD // NS
RB = 512
WN = 8

_mesh = plsc.VectorSubcoreMesh(core_axis_name="c", subcore_axis_name="s",
                               num_cores=NC, num_subcores=NS)
_sc_params = pltpu.CompilerParams(use_tc_tiling_on_sc=False)


NSETS = 2


def _make_agg_body(width, rt, edge_split, dtype, glen, che):
    ne = rt * IDX_B
    assert ne % che == 0 and che % glen == 0
    gpch = che // glen

    def body(x_hbm, src_hbm, dst_hbm, zero_hbm, out_hbm,
             idx_s, idx_d, rows, sg0, sg1, ss0, ss1, acc_sh):
        cid = lax.axis_index("c")
        sid = lax.axis_index("s")
        sem_g = (sg0, sg1)
        sem_s = (ss0, ss1)
        pltpu.sync_copy(zero_hbm.at[pl.ds(sid * ZR, ZR)],
                        acc_sh.at[pl.ds(sid * ZR, ZR)])
        plsc.subcore_barrier()
        base = ((sid * NC + cid) if edge_split else sid) * ne

        rows2d = glen == IDX_B

        def chunk(c, carry):
            off = base + c * che
            if rows2d:
                orow = off // IDX_B
                nrow = che // IDX_B
                if edge_split:
                    pltpu.sync_copy(src_hbm.at[pl.ds(orow, nrow)], idx_s)
                else:
                    pltpu.sync_copy(src_hbm.at[cid, pl.ds(orow, nrow)], idx_s)
                pltpu.sync_copy(dst_hbm.at[pl.ds(orow, nrow)], idx_d)
            else:
                if edge_split:
                    pltpu.sync_copy(src_hbm.at[pl.ds(off, che)], idx_s)
                else:
                    pltpu.sync_copy(src_hbm.at[cid, pl.ds(off, che)], idx_s)
                pltpu.sync_copy(dst_hbm.at[pl.ds(off, che)], idx_d)
            hg = [None] * NSETS
            hs = [None] * NSETS

            def fire_g(g):
                s_ = g % NSETS
                isl = idx_s.at[g] if rows2d else idx_s.at[pl.ds(g * glen, glen)]
                hg[s_] = pltpu.async_copy(x_hbm.at[isl], rows.at[s_], sem_g[s_])

            def fire_s(g):
                s_ = g % NSETS
                dsl = idx_d.at[g] if rows2d else idx_d.at[pl.ds(g * glen, glen)]
                hs[s_] = pltpu.async_copy(rows.at[s_], acc_sh.at[dsl],
                                          sem_s[s_], add=True)

            fire_g(0)
            for g in range(gpch):
                s_ = g % NSETS
                hg[s_].wait()
                fire_s(g)
                if g + 1 < gpch:
                    s2 = (g + 1) % NSETS
                    if hs[s2] is not None:
                        hs[s2].wait()
                        hs[s2] = None
                    fire_g(g + 1)
            for h in hs:
                if h is not None:
                    h.wait()
            return carry

        lax.fori_loop(0, ne // che, chunk, 0)
        plsc.subcore_barrier()
        pltpu.sync_copy(acc_sh.at[pl.ds(sid * ZR, ZR)],
                        out_hbm.at[cid, pl.ds(sid * ZR, ZR)])

    return body


def _make_agg(width, rt, edge_split, dtype=jnp.float32, glen=128, che=1024):
    return pl.kernel(
        _make_agg_body(width, rt, edge_split, dtype, glen, che),
        out_type=jax.ShapeDtypeStruct((NC, N_PAD, width), dtype),
        mesh=_mesh,
        scratch_types=[
            pltpu.VMEM((che // IDX_B, IDX_B) if glen == IDX_B else (che,), jnp.int32),
            pltpu.VMEM((che // IDX_B, IDX_B) if glen == IDX_B else (che,), jnp.int32),
            pltpu.VMEM((NSETS, glen, width), dtype),
            pltpu.SemaphoreType.DMA,
            pltpu.SemaphoreType.DMA,
            pltpu.SemaphoreType.DMA,
            pltpu.SemaphoreType.DMA,
            pltpu.VMEM_SHARED((N_PAD, width), dtype),
        ],
        compiler_params=_sc_params,
    )


_agg4 = _make_agg(WN, RT4, edge_split=True, glen=128, che=1024)
_agg64 = _make_agg(32, RT64, edge_split=False, dtype=jnp.bfloat16, glen=256, che=2560)


_GRID = (N_PAD + RB - 1) // RB


def _leaky(y):
    return jnp.where(y >= 0, y, 0.01 * y)


def _padx_body(f_ref, o_ref):
    f = f_ref[...]
    o_ref[...] = jnp.concatenate(
        [f, jnp.ones((f.shape[0], 1), jnp.float32),
         jnp.zeros((f.shape[0], WN - 4), jnp.float32)], axis=1)


_padx = pl.pallas_call(
    _padx_body,
    grid=(_GRID,),
    in_specs=[pl.BlockSpec((RB, 3), lambda i: (i, 0))],
    out_specs=pl.BlockSpec((RB, WN), lambda i: (i, 0)),
    out_shape=jax.ShapeDtypeStruct((N_PAD, WN), jnp.float32),
)


def _l1_body(p_ref, w_ref, b_ref, y_ref, r_ref):
    p = p_ref[0] + p_ref[1]
    recip = 1.0 / jnp.maximum(p[:, 3], 1.0)
    h = p * recip[:, None]
    y = jnp.dot(h, w_ref[...], preferred_element_type=jnp.float32) + b_ref[...]
    y = _leaky(y).astype(jnp.bfloat16)
    y_ref[0] = y[:, :32]
    y_ref[1] = y[:, 32:]
    r_ref[...] = recip


_l1 = pl.pallas_call(
    _l1_body,
    grid=(_GRID,),
    in_specs=[
        pl.BlockSpec((NC, RB, WN), lambda i: (0, i, 0)),
        pl.BlockSpec((WN, 64), lambda i: (0, 0)),
        pl.BlockSpec((64,), lambda i: (0,)),
    ],
    out_specs=[
        pl.BlockSpec((NC, RB, 32), lambda i: (0, i, 0)),
        pl.BlockSpec((RB,), lambda i: (i,)),
    ],
    out_shape=[
        jax.ShapeDtypeStruct((NC, N_PAD, 32), jnp.bfloat16),
        jax.ShapeDtypeStruct((N_PAD,), jnp.float32),
    ],
)


def _mid_body(a_ref, r_ref, w_ref, b_ref, y_ref):
    a = jnp.concatenate([a_ref[0], a_ref[1]], axis=1).astype(jnp.float32)
    h = a * r_ref[...][:, None]
    y = jnp.dot(h, w_ref[...], preferred_element_type=jnp.float32) + b_ref[...]
    y = _leaky(y).astype(jnp.bfloat16)
    y_ref[0] = y[:, :32]
    y_ref[1] = y[:, 32:]


_mid = pl.pallas_call(
    _mid_body,
    grid=(_GRID,),
    in_specs=[
        pl.BlockSpec((NC, RB, 32), lambda i: (0, i, 0)),
        pl.BlockSpec((RB,), lambda i: (i,)),
        pl.BlockSpec((64, 64), lambda i: (0, 0)),
        pl.BlockSpec((64,), lambda i: (0,)),
    ],
    out_specs=pl.BlockSpec((NC, RB, 32), lambda i: (0, i, 0)),
    out_shape=jax.ShapeDtypeStruct((NC, N_PAD, 32), jnp.bfloat16),
)


def _l4_body(a_ref, r_ref, w_ref, b_ref, w5_ref, y_ref):
    a = jnp.concatenate([a_ref[0], a_ref[1]], axis=1).astype(jnp.float32)
    h = a * r_ref[...][:, None]
    y = jnp.dot(h, w_ref[...], preferred_element_type=jnp.float32) + b_ref[...]
    y = _leaky(y)
    y_ref[...] = jnp.dot(y, w5_ref[...], preferred_element_type=jnp.float32)


_l4 = pl.pallas_call(
    _l4_body,
    grid=(_GRID,),
    in_specs=[
        pl.BlockSpec((NC, RB, 32), lambda i: (0, i, 0)),
        pl.BlockSpec((RB,), lambda i: (i,)),
        pl.BlockSpec((64, 64), lambda i: (0, 0)),
        pl.BlockSpec((64,), lambda i: (0,)),
        pl.BlockSpec((64, WN), lambda i: (0, 0)),
    ],
    out_specs=pl.BlockSpec((RB, WN), lambda i: (i, 0)),
    out_shape=jax.ShapeDtypeStruct((N_PAD, WN), jnp.float32),
)


def _l5_body(p_ref, r_ref, b_ref, o_ref):
    p = p_ref[0] + p_ref[1]
    s = p * r_ref[...][:, None] + b_ref[...]
    o_ref[...] = s[:, :3]


_l5 = pl.pallas_call(
    _l5_body,
    grid=(_GRID,),
    in_specs=[
        pl.BlockSpec((NC, RB, WN), lambda i: (0, i, 0)),
        pl.BlockSpec((RB,), lambda i: (i,)),
        pl.BlockSpec((WN,), lambda i: (0,)),
    ],
    out_specs=pl.BlockSpec((RB, 3), lambda i: (i, 0)),
    out_shape=jax.ShapeDtypeStruct((N, 3), jnp.float32),
)


def kernel(features, edge_index, W1, b1, W2, b2, W3, b3, W4, b4, W5, b5):
    src = edge_index[0]
    dst = edge_index[1]
    pad = E_PAD - E
    srcp = jnp.concatenate([src, jnp.zeros((pad,), jnp.int32)])
    dstp = jnp.concatenate([dst, jnp.full((pad,), DUMMY, jnp.int32)])
    src2 = jnp.stack([srcp, srcp + N_PAD])
    srcp2d = srcp.reshape(EROWS, IDX_B)
    dstp2d = dstp.reshape(EROWS, IDX_B)
    zeros4 = jnp.zeros((N_PAD, WN), jnp.float32)
    zeros32 = jnp.zeros((N_PAD, 32), jnp.bfloat16)
    w1p = jnp.concatenate([W1, jnp.zeros((WN - 3, 64), jnp.float32)], axis=0)
    w5p = jnp.concatenate([W5, jnp.zeros((64, WN - 3), jnp.float32)], axis=1)
    b5p = jnp.concatenate([b5, jnp.zeros((WN - 3,), jnp.float32)])

    x4 = _padx(features)
    part1 = _agg4(x4, srcp2d, dstp2d, zeros4)
    xflat, recip = _l1(part1, w1p, b1)
    for w, b in ((W2, b2), (W3, b3)):
        agg = _agg64(xflat.reshape(2 * N_PAD, 32), src2, dstp, zeros32)
        xflat = _mid(agg, recip, w, b)
    agg = _agg64(xflat.reshape(2 * N_PAD, 32), src2, dstp, zeros32)
    x5 = _l4(agg, recip, W4, b4, w5p)
    part5 = _agg4(x5, srcp2d, dstp2d, zeros4)
    return _l5(part5, recip, b5p)


# speedup vs baseline: 1.1392x; 1.1392x over previous
import functools

import jax
import jax.numpy as jnp
from jax import lax
from jax.experimental import pallas as pl
from jax.experimental.pallas import tpu as pltpu
from jax.experimental.pallas import tpu_sc as plsc

N = 50000
E = 800000
NC = 2
NS = 16
NW = NC * NS
IDX_B = 128
N_PAD = 50048
E_PAD = 819200
EROWS = E_PAD // IDX_B
RT4 = EROWS // NW
RT64 = EROWS // NS
DUMMY = N
CH = 8
ZR = N_PAD // NS
RB = 512
WN = 8

_mesh = plsc.VectorSubcoreMesh(core_axis_name="c", subcore_axis_name="s",
                               num_cores=NC, num_subcores=NS)
_sc_params = pltpu.CompilerParams(use_tc_tiling_on_sc=False)


NSETS = 2


def _make_agg_body(width, rt, edge_split, dtype, glen, che, kp):
    ne = rt * IDX_B
    assert ne % che == 0 and che % (glen * kp) == 0
    gpch = che // (glen * kp)

    def body(x_hbm, src_hbm, dst_hbm, zero_hbm, out_hbm,
             idx_s, idx_d, rows, sg0, sg1, ss0, ss1, acc_sh):
        cid = lax.axis_index("c")
        sid = lax.axis_index("s")
        sem_g = (sg0, sg1)
        sem_s = (ss0, ss1)
        pltpu.sync_copy(zero_hbm.at[pl.ds(sid * ZR, ZR)],
                        acc_sh.at[pl.ds(sid * ZR, ZR)])
        plsc.subcore_barrier()
        base = ((sid * NC + cid) if edge_split else sid) * ne

        rows2d = glen == IDX_B

        def chunk(c, carry):
            off = base + c * che
            if rows2d:
                orow = off // IDX_B
                nrow = che // IDX_B
                if edge_split:
                    pltpu.sync_copy(src_hbm.at[pl.ds(orow, nrow)], idx_s)
                else:
                    pltpu.sync_copy(src_hbm.at[cid, pl.ds(orow, nrow)], idx_s)
                pltpu.sync_copy(dst_hbm.at[pl.ds(orow, nrow)], idx_d)
            else:
                if edge_split:
                    pltpu.sync_copy(src_hbm.at[pl.ds(off, che)], idx_s)
                else:
                    pltpu.sync_copy(src_hbm.at[cid, pl.ds(off, che)], idx_s)
                pltpu.sync_copy(dst_hbm.at[pl.ds(off, che)], idx_d)
            hg = [None] * NSETS
            hs = [None] * NSETS

            def isl(g, k):
                if rows2d:
                    return idx_s.at[g * kp + k]
                return idx_s.at[pl.ds((g * kp + k) * glen, glen)]

            def dsl(g, k):
                if rows2d:
                    return idx_d.at[g * kp + k]
                return idx_d.at[pl.ds((g * kp + k) * glen, glen)]

            def fire_g(g):
                s_ = g % NSETS
                hg[s_] = [pltpu.async_copy(x_hbm.at[isl(g, k)],
                                           rows.at[kp * s_ + k], sem_g[s_])
                          for k in range(kp)]

            def fire_s(g):
                s_ = g % NSETS
                hs[s_] = [pltpu.async_copy(rows.at[kp * s_ + k],
                                           acc_sh.at[dsl(g, k)],
                                           sem_s[s_], add=True)
                          for k in range(kp)]

            fire_g(0)
            for g in range(gpch):
                s_ = g % NSETS
                for h in hg[s_]:
                    h.wait()
                fire_s(g)
                if g + 1 < gpch:
                    s2 = (g + 1) % NSETS
                    if hs[s2] is not None:
                        for h in hs[s2]:
                            h.wait()
                        hs[s2] = None
                    fire_g(g + 1)
            for hl in hs:
                if hl is not None:
                    for h in hl:
                        h.wait()
            return carry

        lax.fori_loop(0, ne // che, chunk, 0)
        plsc.subcore_barrier()
        pltpu.sync_copy(acc_sh.at[pl.ds(sid * ZR, ZR)],
                        out_hbm.at[cid, pl.ds(sid * ZR, ZR)])

    return body


def _make_agg(width, rt, edge_split, dtype=jnp.float32, glen=128, che=1024, kp=1):
    return pl.kernel(
        _make_agg_body(width, rt, edge_split, dtype, glen, che, kp),
        out_type=jax.ShapeDtypeStruct((NC, N_PAD, width), dtype),
        mesh=_mesh,
        scratch_types=[
            pltpu.VMEM((che // IDX_B, IDX_B) if glen == IDX_B else (che,), jnp.int32),
            pltpu.VMEM((che // IDX_B, IDX_B) if glen == IDX_B else (che,), jnp.int32),
            pltpu.VMEM((NSETS * kp, glen, width), dtype),
            pltpu.SemaphoreType.DMA,
            pltpu.SemaphoreType.DMA,
            pltpu.SemaphoreType.DMA,
            pltpu.SemaphoreType.DMA,
            pltpu.VMEM_SHARED((N_PAD, width), dtype),
        ],
        compiler_params=_sc_params,
    )


_agg4 = _make_agg(WN, RT4, edge_split=True, glen=128, che=1024, kp=2)
_agg64 = _make_agg(32, RT64, edge_split=False, dtype=jnp.bfloat16, glen=128, che=5120, kp=4)


_GRID = (N_PAD + RB - 1) // RB


def _leaky(y):
    return jnp.where(y >= 0, y, 0.01 * y)


def _padx_body(f_ref, o_ref):
    f = f_ref[...]
    o_ref[...] = jnp.concatenate(
        [f, jnp.ones((f.shape[0], 1), jnp.float32),
         jnp.zeros((f.shape[0], WN - 4), jnp.float32)], axis=1)


_padx = pl.pallas_call(
    _padx_body,
    grid=(_GRID,),
    in_specs=[pl.BlockSpec((RB, 3), lambda i: (i, 0))],
    out_specs=pl.BlockSpec((RB, WN), lambda i: (i, 0)),
    out_shape=jax.ShapeDtypeStruct((N_PAD, WN), jnp.float32),
)


def _l1_body(p_ref, w_ref, b_ref, y_ref, r_ref):
    p = p_ref[0] + p_ref[1]
    recip = 1.0 / jnp.maximum(p[:, 3], 1.0)
    h = p * recip[:, None]
    y = jnp.dot(h, w_ref[...], preferred_element_type=jnp.float32) + b_ref[...]
    y = _leaky(y).astype(jnp.bfloat16)
    y_ref[0] = y[:, :32]
    y_ref[1] = y[:, 32:]
    r_ref[...] = recip


_l1 = pl.pallas_call(
    _l1_body,
    grid=(_GRID,),
    in_specs=[
        pl.BlockSpec((NC, RB, WN), lambda i: (0, i, 0)),
        pl.BlockSpec((WN, 64), lambda i: (0, 0)),
        pl.BlockSpec((64,), lambda i: (0,)),
    ],
    out_specs=[
        pl.BlockSpec((NC, RB, 32), lambda i: (0, i, 0)),
        pl.BlockSpec((RB,), lambda i: (i,)),
    ],
    out_shape=[
        jax.ShapeDtypeStruct((NC, N_PAD, 32), jnp.bfloat16),
        jax.ShapeDtypeStruct((N_PAD,), jnp.float32),
    ],
)


def _mid_body(a_ref, r_ref, w_ref, b_ref, y_ref):
    a = jnp.concatenate([a_ref[0], a_ref[1]], axis=1).astype(jnp.float32)
    h = a * r_ref[...][:, None]
    y = jnp.dot(h, w_ref[...], preferred_element_type=jnp.float32) + b_ref[...]
    y = _leaky(y).astype(jnp.bfloat16)
    y_ref[0] = y[:, :32]
    y_ref[1] = y[:, 32:]


_mid = pl.pallas_call(
    _mid_body,
    grid=(_GRID,),
    in_specs=[
        pl.BlockSpec((NC, RB, 32), lambda i: (0, i, 0)),
        pl.BlockSpec((RB,), lambda i: (i,)),
        pl.BlockSpec((64, 64), lambda i: (0, 0)),
        pl.BlockSpec((64,), lambda i: (0,)),
    ],
    out_specs=pl.BlockSpec((NC, RB, 32), lambda i: (0, i, 0)),
    out_shape=jax.ShapeDtypeStruct((NC, N_PAD, 32), jnp.bfloat16),
)


def _l4_body(a_ref, r_ref, w_ref, b_ref, w5_ref, y_ref):
    a = jnp.concatenate([a_ref[0], a_ref[1]], axis=1).astype(jnp.float32)
    h = a * r_ref[...][:, None]
    y = jnp.dot(h, w_ref[...], preferred_element_type=jnp.float32) + b_ref[...]
    y = _leaky(y)
    y_ref[...] = jnp.dot(y, w5_ref[...], preferred_element_type=jnp.float32)


_l4 = pl.pallas_call(
    _l4_body,
    grid=(_GRID,),
    in_specs=[
        pl.BlockSpec((NC, RB, 32), lambda i: (0, i, 0)),
        pl.BlockSpec((RB,), lambda i: (i,)),
        pl.BlockSpec((64, 64), lambda i: (0, 0)),
        pl.BlockSpec((64,), lambda i: (0,)),
        pl.BlockSpec((64, WN), lambda i: (0, 0)),
    ],
    out_specs=pl.BlockSpec((RB, WN), lambda i: (i, 0)),
    out_shape=jax.ShapeDtypeStruct((N_PAD, WN), jnp.float32),
)


def _l5_body(p_ref, r_ref, b_ref, o_ref):
    p = p_ref[0] + p_ref[1]
    s = p * r_ref[...][:, None] + b_ref[...]
    o_ref[...] = s[:, :3]


_l5 = pl.pallas_call(
    _l5_body,
    grid=(_GRID,),
    in_specs=[
        pl.BlockSpec((NC, RB, WN), lambda i: (0, i, 0)),
        pl.BlockSpec((RB,), lambda i: (i,)),
        pl.BlockSpec((WN,), lambda i: (0,)),
    ],
    out_specs=pl.BlockSpec((RB, 3), lambda i: (i, 0)),
    out_shape=jax.ShapeDtypeStruct((N, 3), jnp.float32),
)


def kernel(features, edge_index, W1, b1, W2, b2, W3, b3, W4, b4, W5, b5):
    src = edge_index[0]
    dst = edge_index[1]
    pad = E_PAD - E
    srcp = jnp.concatenate([src, jnp.zeros((pad,), jnp.int32)])
    dstp = jnp.concatenate([dst, jnp.full((pad,), DUMMY, jnp.int32)])
    srcp2d = srcp.reshape(EROWS, IDX_B)
    dstp2d = dstp.reshape(EROWS, IDX_B)
    src2 = jnp.stack([srcp2d, srcp2d + N_PAD])
    zeros4 = jnp.zeros((N_PAD, WN), jnp.float32)
    zeros32 = jnp.zeros((N_PAD, 32), jnp.bfloat16)
    w1p = jnp.concatenate([W1, jnp.zeros((WN - 3, 64), jnp.float32)], axis=0)
    w5p = jnp.concatenate([W5, jnp.zeros((64, WN - 3), jnp.float32)], axis=1)
    b5p = jnp.concatenate([b5, jnp.zeros((WN - 3,), jnp.float32)])

    x4 = _padx(features)
    part1 = _agg4(x4, srcp2d, dstp2d, zeros4)
    xflat, recip = _l1(part1, w1p, b1)
    for w, b in ((W2, b2), (W3, b3)):
        agg = _agg64(xflat.reshape(2 * N_PAD, 32), src2, dstp2d, zeros32)
        xflat = _mid(agg, recip, w, b)
    agg = _agg64(xflat.reshape(2 * N_PAD, 32), src2, dstp2d, zeros32)
    x5 = _l4(agg, recip, W4, b4, w5p)
    part5 = _agg4(x5, srcp2d, dstp2d, zeros4)
    return _l5(part5, recip, b5p)


# speedup vs baseline: 1.2157x; 1.0672x over previous
import functools

import jax
import jax.numpy as jnp
from jax import lax
from jax.experimental import pallas as pl
from jax.experimental.pallas import tpu as pltpu
from jax.experimental.pallas import tpu_sc as plsc

N = 50000
E = 800000
NC = 2
NS = 16
NW = NC * NS
IDX_B = 128
N_PAD = 50048
E_PAD = 819200
EROWS = E_PAD // IDX_B
RT4 = EROWS // NW
RT64 = EROWS // NS
DUMMY = N
CH = 8
ZR = N_PAD // NS
RB = 512
WN = 8

_mesh = plsc.VectorSubcoreMesh(core_axis_name="c", subcore_axis_name="s",
                               num_cores=NC, num_subcores=NS)
_sc_params = pltpu.CompilerParams(use_tc_tiling_on_sc=False)


NSETS = 2


def _make_agg_body(width, rt, edge_split, dtype, glen, che, kp):
    ne = rt * IDX_B
    assert ne % che == 0 and che % (glen * kp) == 0
    gpch = che // (glen * kp)

    def body(x_hbm, src_hbm, dst_hbm, zero_hbm, out_hbm,
             idx_s, idx_d, rows, sg0, sg1, ss0, ss1, acc_sh):
        cid = lax.axis_index("c")
        sid = lax.axis_index("s")
        sem_g = (sg0, sg1)
        sem_s = (ss0, ss1)
        pltpu.sync_copy(zero_hbm.at[pl.ds(sid * ZR, ZR)],
                        acc_sh.at[pl.ds(sid * ZR, ZR)])
        plsc.subcore_barrier()
        base = ((sid * NC + cid) if edge_split else sid) * ne

        rows2d = glen == IDX_B

        def chunk(c, carry):
            off = base + c * che
            if rows2d:
                orow = off // IDX_B
                nrow = che // IDX_B
                if edge_split:
                    pltpu.sync_copy(src_hbm.at[pl.ds(orow, nrow)], idx_s)
                else:
                    pltpu.sync_copy(src_hbm.at[cid, pl.ds(orow, nrow)], idx_s)
                pltpu.sync_copy(dst_hbm.at[pl.ds(orow, nrow)], idx_d)
            else:
                if edge_split:
                    pltpu.sync_copy(src_hbm.at[pl.ds(off, che)], idx_s)
                else:
                    pltpu.sync_copy(src_hbm.at[cid, pl.ds(off, che)], idx_s)
                pltpu.sync_copy(dst_hbm.at[pl.ds(off, che)], idx_d)
            hg = [None] * NSETS
            hs = [None] * NSETS

            def isl(g, k):
                if rows2d:
                    return idx_s.at[g * kp + k]
                return idx_s.at[pl.ds((g * kp + k) * glen, glen)]

            def dsl(g, k):
                if rows2d:
                    return idx_d.at[g * kp + k]
                return idx_d.at[pl.ds((g * kp + k) * glen, glen)]

            def fire_g(g):
                s_ = g % NSETS
                hg[s_] = [pltpu.async_copy(x_hbm.at[isl(g, k)],
                                           rows.at[kp * s_ + k], sem_g[s_])
                          for k in range(kp)]

            def fire_s(g):
                s_ = g % NSETS
                hs[s_] = [pltpu.async_copy(rows.at[kp * s_ + k],
                                           acc_sh.at[dsl(g, k)],
                                           sem_s[s_], add=True)
                          for k in range(kp)]

            fire_g(0)
            for g in range(gpch):
                s_ = g % NSETS
                for h in hg[s_]:
                    h.wait()
                fire_s(g)
                if g + 1 < gpch:
                    s2 = (g + 1) % NSETS
                    if hs[s2] is not None:
                        for h in hs[s2]:
                            h.wait()
                        hs[s2] = None
                    fire_g(g + 1)
            for hl in hs:
                if hl is not None:
                    for h in hl:
                        h.wait()
            return carry

        lax.fori_loop(0, ne // che, chunk, 0)
        plsc.subcore_barrier()
        pltpu.sync_copy(acc_sh.at[pl.ds(sid * ZR, ZR)],
                        out_hbm.at[cid, pl.ds(sid * ZR, ZR)])

    return body


def _make_agg(width, rt, edge_split, dtype=jnp.float32, glen=128, che=1024, kp=1):
    return pl.kernel(
        _make_agg_body(width, rt, edge_split, dtype, glen, che, kp),
        out_type=jax.ShapeDtypeStruct((NC, N_PAD, width), dtype),
        mesh=_mesh,
        scratch_types=[
            pltpu.VMEM((che // IDX_B, IDX_B) if glen == IDX_B else (che,), jnp.int32),
            pltpu.VMEM((che // IDX_B, IDX_B) if glen == IDX_B else (che,), jnp.int32),
            pltpu.VMEM((NSETS * kp, glen, width), dtype),
            pltpu.SemaphoreType.DMA,
            pltpu.SemaphoreType.DMA,
            pltpu.SemaphoreType.DMA,
            pltpu.SemaphoreType.DMA,
            pltpu.VMEM_SHARED((N_PAD, width), dtype),
        ],
        compiler_params=_sc_params,
    )


_agg4 = _make_agg(WN, RT4, edge_split=True, glen=128, che=5120, kp=4)
_agg64 = _make_agg(32, RT64, edge_split=False, dtype=jnp.bfloat16, glen=128, che=5120, kp=8)


_GRID = (N_PAD + RB - 1) // RB


def _leaky(y):
    return jnp.where(y >= 0, y, 0.01 * y)


def _padx_body(f_ref, o_ref):
    f = f_ref[...]
    o_ref[...] = jnp.concatenate(
        [f, jnp.ones((f.shape[0], 1), jnp.float32),
         jnp.zeros((f.shape[0], WN - 4), jnp.float32)], axis=1)


_padx = pl.pallas_call(
    _padx_body,
    grid=(_GRID,),
    in_specs=[pl.BlockSpec((RB, 3), lambda i: (i, 0))],
    out_specs=pl.BlockSpec((RB, WN), lambda i: (i, 0)),
    out_shape=jax.ShapeDtypeStruct((N_PAD, WN), jnp.float32),
)


def _l1_body(p_ref, w_ref, b_ref, y_ref, r_ref):
    p = p_ref[0] + p_ref[1]
    recip = 1.0 / jnp.maximum(p[:, 3], 1.0)
    h = p * recip[:, None]
    y = jnp.dot(h, w_ref[...], preferred_element_type=jnp.float32) + b_ref[...]
    y = _leaky(y).astype(jnp.bfloat16)
    y_ref[0] = y[:, :32]
    y_ref[1] = y[:, 32:]
    r_ref[...] = recip


_l1 = pl.pallas_call(
    _l1_body,
    grid=(_GRID,),
    in_specs=[
        pl.BlockSpec((NC, RB, WN), lambda i: (0, i, 0)),
        pl.BlockSpec((WN, 64), lambda i: (0, 0)),
        pl.BlockSpec((64,), lambda i: (0,)),
    ],
    out_specs=[
        pl.BlockSpec((NC, RB, 32), lambda i: (0, i, 0)),
        pl.BlockSpec((RB,), lambda i: (i,)),
    ],
    out_shape=[
        jax.ShapeDtypeStruct((NC, N_PAD, 32), jnp.bfloat16),
        jax.ShapeDtypeStruct((N_PAD,), jnp.float32),
    ],
)


def _mid_body(a_ref, r_ref, w_ref, b_ref, y_ref):
    a = jnp.concatenate([a_ref[0], a_ref[1]], axis=1).astype(jnp.float32)
    h = a * r_ref[...][:, None]
    y = jnp.dot(h, w_ref[...], preferred_element_type=jnp.float32) + b_ref[...]
    y = _leaky(y).astype(jnp.bfloat16)
    y_ref[0] = y[:, :32]
    y_ref[1] = y[:, 32:]


_mid = pl.pallas_call(
    _mid_body,
    grid=(_GRID,),
    in_specs=[
        pl.BlockSpec((NC, RB, 32), lambda i: (0, i, 0)),
        pl.BlockSpec((RB,), lambda i: (i,)),
        pl.BlockSpec((64, 64), lambda i: (0, 0)),
        pl.BlockSpec((64,), lambda i: (0,)),
    ],
    out_specs=pl.BlockSpec((NC, RB, 32), lambda i: (0, i, 0)),
    out_shape=jax.ShapeDtypeStruct((NC, N_PAD, 32), jnp.bfloat16),
)


def _l4_body(a_ref, r_ref, w_ref, b_ref, w5_ref, y_ref):
    a = jnp.concatenate([a_ref[0], a_ref[1]], axis=1).astype(jnp.float32)
    h = a * r_ref[...][:, None]
    y = jnp.dot(h, w_ref[...], preferred_element_type=jnp.float32) + b_ref[...]
    y = _leaky(y)
    y_ref[...] = jnp.dot(y, w5_ref[...], preferred_element_type=jnp.float32)


_l4 = pl.pallas_call(
    _l4_body,
    grid=(_GRID,),
    in_specs=[
        pl.BlockSpec((NC, RB, 32), lambda i: (0, i, 0)),
        pl.BlockSpec((RB,), lambda i: (i,)),
        pl.BlockSpec((64, 64), lambda i: (0, 0)),
        pl.BlockSpec((64,), lambda i: (0,)),
        pl.BlockSpec((64, WN), lambda i: (0, 0)),
    ],
    out_specs=pl.BlockSpec((RB, WN), lambda i: (i, 0)),
    out_shape=jax.ShapeDtypeStruct((N_PAD, WN), jnp.float32),
)


def _l5_body(p_ref, r_ref, b_ref, o_ref):
    p = p_ref[0] + p_ref[1]
    s = p * r_ref[...][:, None] + b_ref[...]
    o_ref[...] = s[:, :3]


_l5 = pl.pallas_call(
    _l5_body,
    grid=(_GRID,),
    in_specs=[
        pl.BlockSpec((NC, RB, WN), lambda i: (0, i, 0)),
        pl.BlockSpec((RB,), lambda i: (i,)),
        pl.BlockSpec((WN,), lambda i: (0,)),
    ],
    out_specs=pl.BlockSpec((RB, 3), lambda i: (i, 0)),
    out_shape=jax.ShapeDtypeStruct((N, 3), jnp.float32),
)


def kernel(features, edge_index, W1, b1, W2, b2, W3, b3, W4, b4, W5, b5):
    src = edge_index[0]
    dst = edge_index[1]
    pad = E_PAD - E
    srcp = jnp.concatenate([src, jnp.zeros((pad,), jnp.int32)])
    dstp = jnp.concatenate([dst, jnp.full((pad,), DUMMY, jnp.int32)])
    srcp2d = srcp.reshape(EROWS, IDX_B)
    dstp2d = dstp.reshape(EROWS, IDX_B)
    src2 = jnp.stack([srcp2d, srcp2d + N_PAD])
    zeros4 = jnp.zeros((N_PAD, WN), jnp.float32)
    zeros32 = jnp.zeros((N_PAD, 32), jnp.bfloat16)
    w1p = jnp.concatenate([W1, jnp.zeros((WN - 3, 64), jnp.float32)], axis=0)
    w5p = jnp.concatenate([W5, jnp.zeros((64, WN - 3), jnp.float32)], axis=1)
    b5p = jnp.concatenate([b5, jnp.zeros((WN - 3,), jnp.float32)])

    x4 = _padx(features)
    part1 = _agg4(x4, srcp2d, dstp2d, zeros4)
    xflat, recip = _l1(part1, w1p, b1)
    for w, b in ((W2, b2), (W3, b3)):
        agg = _agg64(xflat.reshape(2 * N_PAD, 32), src2, dstp2d, zeros32)
        xflat = _mid(agg, recip, w, b)
    agg = _agg64(xflat.reshape(2 * N_PAD, 32), src2, dstp2d, zeros32)
    x5 = _l4(agg, recip, W4, b4, w5p)
    part5 = _agg4(x5, srcp2d, dstp2d, zeros4)
    return _l5(part5, recip, b5p)
